# Initial kernel scaffold; baseline (speedup 1.0000x reference)
import functools

import jax
import jax.numpy as jnp
from jax import lax
from jax.experimental import pallas as pl
from jax.experimental.pallas import tpu as pltpu
from jax.experimental.pallas import tpu_sc as plsc

B, C = 1024, 50
POS = B * C
QDIMS = (10, 10, 10, 10, 10, 10, 30)
DDIMS = (10, 10, 10, 10, 10, 5, 10)
NW = 32
DPW = POS // NW
QPW = B // NW
BB = 8
GRID = B // BB


def _sc_gather(d_tables, q_tables, ddiscT, qdiscT):
  mesh = plsc.VectorSubcoreMesh(core_axis_name="c", subcore_axis_name="s")
  out_type = (
      [jax.ShapeDtypeStruct((POS, d), jnp.float32) for d in DDIMS]
      + [jax.ShapeDtypeStruct((B, d), jnp.float32) for d in QDIMS]
  )
  scratch_types = (
      [pltpu.VMEM((DPW,), jnp.int32) for _ in range(7)]
      + [pltpu.VMEM((DPW, d), jnp.float32) for d in DDIMS]
      + [pltpu.VMEM((QPW,), jnp.int32) for _ in range(7)]
      + [pltpu.VMEM((QPW, d), jnp.float32) for d in QDIMS]
      + [pltpu.SemaphoreType.DMA]
  )

  @functools.partial(pl.kernel, mesh=mesh, out_type=out_type,
                     scratch_types=scratch_types)
  def gather_kernel(*refs):
    dt = refs[0:7]
    qt = refs[7:14]
    ddisc_ref = refs[14]
    qdisc_ref = refs[15]
    douts = refs[16:23]
    qouts = refs[23:30]
    didx = refs[30:37]
    dbuf = refs[37:44]
    qidx = refs[44:51]
    qbuf = refs[51:58]
    sem = refs[58]

    wid = lax.axis_index("s") * 2 + lax.axis_index("c")
    dbase = wid * DPW
    qbase = wid * QPW

    for f in range(7):
      pltpu.sync_copy(ddisc_ref.at[pl.ds(f * POS + dbase, DPW)], didx[f])
      pltpu.sync_copy(qdisc_ref.at[pl.ds(f * B + qbase, QPW)], qidx[f])

    n_full = DPW // 128
    for f in range(7):
      def chunk(j, carry, f=f):
        off = pl.multiple_of(j * 128, 128)
        pltpu.async_copy(dt[f].at[didx[f].at[pl.ds(off, 128)]],
                         dbuf[f].at[pl.ds(off, 128), :], sem)
        return carry
      lax.fori_loop(0, n_full, chunk, 0)
      rem = DPW - n_full * 128
      if rem:
        pltpu.async_copy(dt[f].at[didx[f].at[pl.ds(n_full * 128, rem)]],
                         dbuf[f].at[pl.ds(n_full * 128, rem), :], sem)
    for f in range(7):
      pltpu.async_copy(qt[f].at[qidx[f]], qbuf[f], sem)

    for f in range(7):
      pltpu.make_async_copy(dt[f].at[didx[f]], dbuf[f], sem).wait()
    for f in range(7):
      pltpu.make_async_copy(qt[f].at[qidx[f]], qbuf[f], sem).wait()

    for f in range(7):
      pltpu.sync_copy(dbuf[f], douts[f].at[pl.ds(dbase, DPW), :])
    for f in range(7):
      pltpu.sync_copy(qbuf[f], qouts[f].at[pl.ds(qbase, QPW), :])

  return gather_kernel(*d_tables, *q_tables, ddiscT, qdiscT)


def _tc_body(*refs):
  (idxs, ratings, qcont, qdisc, dcontF, qdcontF, ddiscF,
   de0, de1, de2, de3, de4, de5, de6,
   qe0, qe1, qe2, qe3, qe4, qe5, qe6,
   Wqc, bqc, Wdc, bdc, Wqd, bqd,
   wq0, wq1, wq2, wq3, wq4, wq5, wq6, bqdisc,
   wd0, wd1, wd2, wd3, wd4, wd5, wd6, bddisc,
   WattnA, WattnB, Wattn, battn, WfinT, bfin, acc) = refs
  des = (de0, de1, de2, de3, de4, de5, de6)
  qes = (qe0, qe1, qe2, qe3, qe4, qe5, qe6)
  wqs = (wq0, wq1, wq2, wq3, wq4, wq5, wq6)
  wds = (wd0, wd1, wd2, wd3, wd4, wd5, wd6)
  i = pl.program_id(0)
  f32 = jnp.float32
  dot = lambda a, b: jnp.dot(a, b, preferred_element_type=f32)

  qdisc_i = qdisc[...]
  qacc = bqdisc[...]
  for f in range(7):
    m = (qdisc_i[:, f:f + 1] != 0).astype(f32)
    qacc = qacc + dot(qes[f][...] * m, wqs[f][...])
  qdisc_h = jnp.tanh(qacc)
  qcont_h = jnp.tanh(dot(qcont[...], Wqc[...]) + bqc[...])
  u_q8 = jnp.tanh(dot(qcont_h, WattnA[...]) + dot(qdisc_h, WattnB[...])
                  + battn[...])

  ddisc_i = ddiscF[...]
  dacc = bddisc[...]
  for f in range(7):
    m = (ddisc_i[:, f:f + 1] != 0).astype(f32)
    dacc = dacc + dot(des[f][...] * m, wds[f][...])
  ddisc_h = jnp.tanh(dacc)
  dcont_h = jnp.tanh(dot(dcontF[...], Wdc[...]) + bdc[...])
  u_d = jnp.tanh(dot(dcont_h, WattnA[...]) + dot(ddisc_h, WattnB[...])
                 + battn[...])
  qdcont_h = jnp.tanh(dot(qdcontF[...], Wqd[...]) + bqd[...])
  u_qd = jnp.tanh(dot(qdcont_h, Wattn[...]) + battn[...])

  r = lax.broadcasted_iota(jnp.int32, (BB * C, BB), 0) // C
  cix = lax.broadcasted_iota(jnp.int32, (BB * C, BB), 1)
  S = (r == cix).astype(f32)
  u_q = dot(S, u_q8)

  rdot = lambda a, b: jnp.sum(a * b, axis=1, keepdims=True)
  g_qq = rdot(u_q, u_q)
  g_qd = rdot(u_q, u_d)
  g_qqd = rdot(u_q, u_qd)
  g_dd = rdot(u_d, u_d)
  g_dqd = rdot(u_d, u_qd)
  g_qdqd = rdot(u_qd, u_qd)

  s0 = jnp.zeros_like(g_qq)
  s1 = jnp.zeros_like(g_qq)
  s2 = jnp.zeros_like(g_qq)
  for a, b, c in ((g_qq, g_qd, g_qqd), (g_qd, g_dd, g_dqd),
                  (g_qqd, g_dqd, g_qdqd)):
    m = jnp.maximum(jnp.maximum(a, b), c)
    ea = jnp.exp(a - m)
    eb = jnp.exp(b - m)
    ec = jnp.exp(c - m)
    inv = 1.0 / (ea + eb + ec)
    s0 = s0 + ea * inv
    s1 = s1 + eb * inv
    s2 = s2 + ec * inv

  wfin = WfinT[...]
  v_q = rdot(u_q, jnp.broadcast_to(wfin, u_q.shape))
  v_d = rdot(u_d, jnp.broadcast_to(wfin, u_d.shape))
  v_qd = rdot(u_qd, jnp.broadcast_to(wfin, u_qd.shape))
  score = s0 * v_q + s1 * v_d + s2 * v_qd + bfin[...]

  rr = lax.broadcasted_iota(jnp.int32, (BB, BB * C), 0)
  cc = lax.broadcasted_iota(jnp.int32, (BB, BB * C), 1)
  ST = (cc // C == rr).astype(f32)
  pcol = lax.broadcasted_iota(jnp.int32, (BB * C, C), 0) % C
  ccol = lax.broadcasted_iota(jnp.int32, (BB * C, C), 1)
  P = jnp.broadcast_to(score, (BB * C, C)) * (pcol == ccol).astype(f32)
  scores2d = dot(ST, P)

  mask2 = (idxs[...] != 0).astype(f32)
  sc = scores2d * mask2
  mx = jnp.max(sc, axis=1, keepdims=True)
  lse = jnp.log(jnp.sum(jnp.exp(sc - mx), axis=1, keepdims=True))
  lsm = sc - mx - lse
  part = jnp.sum(-lsm * ratings[...].astype(f32) * mask2) * (1.0 / B)

  @pl.when(i == 0)
  def _():
    acc[:, :] = jnp.zeros_like(acc)
  acc[:, :] = acc[:, :] + part.reshape(1, 1)


def kernel(candi_doc_idxs, candi_doc_ratings, candi_doc_qcont_features,
           candi_doc_dcont_features, candi_doc_qdcont_features,
           candi_doc_qdiscrete_features, candi_doc_ddiscrete_features,
           q_tables, d_tables, Wqc, bqc, Wdc, bdc, Wqd, bqd,
           Wqdisc, bqdisc, Wddisc, bddisc, Wattn, battn, Wfin, bfin):
  idxs = candi_doc_idxs
  ratings = candi_doc_ratings
  qcont = candi_doc_qcont_features
  dcontF = candi_doc_dcont_features.reshape(POS, -1)
  qdcontF = candi_doc_qdcont_features.reshape(POS, -1)
  qdisc = candi_doc_qdiscrete_features.astype(jnp.int32)
  ddisc = candi_doc_ddiscrete_features.astype(jnp.int32)
  ddiscF = ddisc.reshape(POS, 7)
  ddiscT = jnp.transpose(ddisc, (2, 0, 1)).reshape(7 * POS)
  qdiscT = jnp.transpose(qdisc, (1, 0)).reshape(7 * B)

  gathered = _sc_gather(tuple(d_tables), tuple(q_tables), ddiscT, qdiscT)
  d_embs = gathered[0:7]
  q_embs = gathered[7:14]

  qoff, doff = [0], [0]
  for d in QDIMS:
    qoff.append(qoff[-1] + d)
  for d in DDIMS:
    doff.append(doff[-1] + d)
  wqs = [Wqdisc[qoff[f]:qoff[f + 1]] for f in range(7)]
  wds = [Wddisc[doff[f]:doff[f + 1]] for f in range(7)]
  r2 = lambda v: v.reshape(1, -1)

  grid_spec = pl.GridSpec(
      grid=(GRID,),
      in_specs=[
          pl.BlockSpec((BB, C), lambda i: (i, 0)),
          pl.BlockSpec((BB, C), lambda i: (i, 0)),
          pl.BlockSpec((BB, 15), lambda i: (i, 0)),
          pl.BlockSpec((BB, 7), lambda i: (i, 0)),
          pl.BlockSpec((BB * C, 25), lambda i: (i, 0)),
          pl.BlockSpec((BB * C, 20), lambda i: (i, 0)),
          pl.BlockSpec((BB * C, 7), lambda i: (i, 0)),
      ]
      + [pl.BlockSpec((BB * C, d), lambda i: (i, 0)) for d in DDIMS]
      + [pl.BlockSpec((BB, d), lambda i: (i, 0)) for d in QDIMS]
      + [pl.BlockSpec(w.shape, lambda i: (0, 0)) for w in
         (Wqc, r2(bqc), Wdc, r2(bdc), Wqd, r2(bqd))]
      + [pl.BlockSpec(w.shape, lambda i: (0, 0)) for w in wqs]
      + [pl.BlockSpec(r2(bqdisc).shape, lambda i: (0, 0))]
      + [pl.BlockSpec(w.shape, lambda i: (0, 0)) for w in wds]
      + [pl.BlockSpec(r2(bddisc).shape, lambda i: (0, 0))]
      + [pl.BlockSpec((64, 128), lambda i: (0, 0)),
         pl.BlockSpec((64, 128), lambda i: (0, 0)),
         pl.BlockSpec((128, 128), lambda i: (0, 0)),
         pl.BlockSpec((1, 128), lambda i: (0, 0)),
         pl.BlockSpec((1, 128), lambda i: (0, 0)),
         pl.BlockSpec((1, 1), lambda i: (0, 0))],
      out_specs=pl.BlockSpec((1, 1), lambda i: (0, 0)),
  )

  res = pl.pallas_call(
      _tc_body,
      grid_spec=grid_spec,
      out_shape=jax.ShapeDtypeStruct((1, 1), jnp.float32),
      compiler_params=pltpu.CompilerParams(
          dimension_semantics=("arbitrary",)),
  )(idxs, ratings, qcont, qdisc, dcontF, qdcontF, ddiscF,
    *d_embs, *q_embs,
    Wqc, r2(bqc), Wdc, r2(bdc), Wqd, r2(bqd),
    *wqs, r2(bqdisc), *wds, r2(bddisc),
    Wattn[:64], Wattn[64:], Wattn, r2(battn), Wfin.reshape(1, -1),
    bfin.reshape(1, 1))
  return res[0, 0]


# speedup vs baseline: 2.3382x; 2.3382x over previous
import functools

import jax
import jax.numpy as jnp
from jax import lax
from jax.experimental import pallas as pl
from jax.experimental.pallas import tpu as pltpu
from jax.experimental.pallas import tpu_sc as plsc

B, C = 1024, 50
POS = B * C
QDIMS = (10, 10, 10, 10, 10, 10, 30)
DDIMS = (10, 10, 10, 10, 10, 5, 10)
NW = 32
DPW = POS // NW
HALVES = 2
HPW = DPW // HALVES
QPW = B // NW
BB = 8
GRID = B // BB


def _sc_gather(d_tables, q_tables, ddiscT, qdiscT):
  mesh = plsc.VectorSubcoreMesh(core_axis_name="c", subcore_axis_name="s")
  out_type = (
      [jax.ShapeDtypeStruct((POS, d), jnp.float32) for d in DDIMS]
      + [jax.ShapeDtypeStruct((B, d), jnp.float32) for d in QDIMS]
  )
  scratch_types = (
      [pltpu.VMEM((HPW,), jnp.int32) for _ in range(7)]
      + [pltpu.VMEM((HPW, d), jnp.float32) for d in DDIMS]
      + [pltpu.VMEM((QPW,), jnp.int32) for _ in range(7)]
      + [pltpu.VMEM((QPW, d), jnp.float32) for d in QDIMS]
      + [pltpu.SemaphoreType.DMA]
  )

  @functools.partial(pl.kernel, mesh=mesh, out_type=out_type,
                     scratch_types=scratch_types,
                     compiler_params=pltpu.CompilerParams(
                         use_tc_tiling_on_sc=False))
  def gather_kernel(*refs):
    dt = refs[0:7]
    qt = refs[7:14]
    ddisc_ref = refs[14]
    qdisc_ref = refs[15]
    douts = refs[16:23]
    qouts = refs[23:30]
    didx = refs[30:37]
    dbuf = refs[37:44]
    qidx = refs[44:51]
    qbuf = refs[51:58]
    sem = refs[58]

    wid = lax.axis_index("s") * 2 + lax.axis_index("c")
    qbase = wid * QPW

    for f in range(7):
      pltpu.sync_copy(qdisc_ref.at[pl.ds(f * B + qbase, QPW)], qidx[f])
    for f in range(7):
      pltpu.async_copy(qt[f].at[qidx[f]], qbuf[f], sem)
    for f in range(7):
      pltpu.make_async_copy(qt[f].at[qidx[f]], qbuf[f], sem).wait()
    for f in range(7):
      pltpu.sync_copy(qbuf[f], qouts[f].at[pl.ds(qbase, QPW), :])

    n_full = HPW // 128
    rem = HPW - n_full * 128
    for h in range(HALVES):
      hbase = wid * DPW + h * HPW
      for f in range(7):
        pltpu.sync_copy(ddisc_ref.at[pl.ds(f * POS + hbase, HPW)], didx[f])
      for f in range(7):
        def chunk(j, carry, f=f):
          off = pl.multiple_of(j * 128, 128)
          pltpu.async_copy(dt[f].at[didx[f].at[pl.ds(off, 128)]],
                           dbuf[f].at[pl.ds(off, 128), :], sem)
          return carry
        lax.fori_loop(0, n_full, chunk, 0)
        if rem:
          pltpu.async_copy(dt[f].at[didx[f].at[pl.ds(n_full * 128, rem)]],
                           dbuf[f].at[pl.ds(n_full * 128, rem), :], sem)
      for f in range(7):
        pltpu.make_async_copy(dt[f].at[didx[f]], dbuf[f], sem).wait()
      for f in range(7):
        pltpu.sync_copy(dbuf[f], douts[f].at[pl.ds(hbase, HPW), :])

  return gather_kernel(*d_tables, *q_tables, ddiscT, qdiscT)


def _tc_body(*refs):
  (idxs, ratings, qcont, qdisc, dcontF, qdcontF, ddiscF,
   de0, de1, de2, de3, de4, de5, de6,
   qe0, qe1, qe2, qe3, qe4, qe5, qe6,
   Wqc, bqc, Wdc, bdc, Wqd, bqd,
   wq0, wq1, wq2, wq3, wq4, wq5, wq6, bqdisc,
   wd0, wd1, wd2, wd3, wd4, wd5, wd6, bddisc,
   WattnA, WattnB, Wattn, battn, WfinT, bfin, acc) = refs
  des = (de0, de1, de2, de3, de4, de5, de6)
  qes = (qe0, qe1, qe2, qe3, qe4, qe5, qe6)
  wqs = (wq0, wq1, wq2, wq3, wq4, wq5, wq6)
  wds = (wd0, wd1, wd2, wd3, wd4, wd5, wd6)
  i = pl.program_id(0)
  f32 = jnp.float32
  dot = lambda a, b: jnp.dot(a, b, preferred_element_type=f32)

  qdisc_i = qdisc[...]
  qacc = bqdisc[...]
  for f in range(7):
    m = (qdisc_i[:, f:f + 1] != 0).astype(f32)
    qacc = qacc + dot(qes[f][...] * m, wqs[f][...])
  qdisc_h = jnp.tanh(qacc)
  qcont_h = jnp.tanh(dot(qcont[...], Wqc[...]) + bqc[...])
  u_q8 = jnp.tanh(dot(qcont_h, WattnA[...]) + dot(qdisc_h, WattnB[...])
                  + battn[...])

  ddisc_i = ddiscF[...]
  dacc = bddisc[...]
  for f in range(7):
    m = (ddisc_i[:, f:f + 1] != 0).astype(f32)
    dacc = dacc + dot(des[f][...] * m, wds[f][...])
  ddisc_h = jnp.tanh(dacc)
  dcont_h = jnp.tanh(dot(dcontF[...], Wdc[...]) + bdc[...])
  u_d = jnp.tanh(dot(dcont_h, WattnA[...]) + dot(ddisc_h, WattnB[...])
                 + battn[...])
  qdcont_h = jnp.tanh(dot(qdcontF[...], Wqd[...]) + bqd[...])
  u_qd = jnp.tanh(dot(qdcont_h, Wattn[...]) + battn[...])

  r = lax.broadcasted_iota(jnp.int32, (BB * C, BB), 0) // C
  cix = lax.broadcasted_iota(jnp.int32, (BB * C, BB), 1)
  S = (r == cix).astype(f32)
  u_q = dot(S, u_q8)

  rdot = lambda a, b: jnp.sum(a * b, axis=1, keepdims=True)
  g_qq = rdot(u_q, u_q)
  g_qd = rdot(u_q, u_d)
  g_qqd = rdot(u_q, u_qd)
  g_dd = rdot(u_d, u_d)
  g_dqd = rdot(u_d, u_qd)
  g_qdqd = rdot(u_qd, u_qd)

  s0 = jnp.zeros_like(g_qq)
  s1 = jnp.zeros_like(g_qq)
  s2 = jnp.zeros_like(g_qq)
  for a, b, c in ((g_qq, g_qd, g_qqd), (g_qd, g_dd, g_dqd),
                  (g_qqd, g_dqd, g_qdqd)):
    m = jnp.maximum(jnp.maximum(a, b), c)
    ea = jnp.exp(a - m)
    eb = jnp.exp(b - m)
    ec = jnp.exp(c - m)
    inv = 1.0 / (ea + eb + ec)
    s0 = s0 + ea * inv
    s1 = s1 + eb * inv
    s2 = s2 + ec * inv

  wfin = WfinT[...]
  v_q = rdot(u_q, jnp.broadcast_to(wfin, u_q.shape))
  v_d = rdot(u_d, jnp.broadcast_to(wfin, u_d.shape))
  v_qd = rdot(u_qd, jnp.broadcast_to(wfin, u_qd.shape))
  score = s0 * v_q + s1 * v_d + s2 * v_qd + bfin[...]

  rr = lax.broadcasted_iota(jnp.int32, (BB, BB * C), 0)
  cc = lax.broadcasted_iota(jnp.int32, (BB, BB * C), 1)
  ST = (cc // C == rr).astype(f32)
  pcol = lax.broadcasted_iota(jnp.int32, (BB * C, C), 0) % C
  ccol = lax.broadcasted_iota(jnp.int32, (BB * C, C), 1)
  P = jnp.broadcast_to(score, (BB * C, C)) * (pcol == ccol).astype(f32)
  scores2d = dot(ST, P)

  mask2 = (idxs[...] != 0).astype(f32)
  sc = scores2d * mask2
  mx = jnp.max(sc, axis=1, keepdims=True)
  lse = jnp.log(jnp.sum(jnp.exp(sc - mx), axis=1, keepdims=True))
  lsm = sc - mx - lse
  part = jnp.sum(-lsm * ratings[...].astype(f32) * mask2) * (1.0 / B)

  @pl.when(i == 0)
  def _():
    acc[:, :] = jnp.zeros_like(acc)
  acc[:, :] = acc[:, :] + part.reshape(1, 1)


def kernel(candi_doc_idxs, candi_doc_ratings, candi_doc_qcont_features,
           candi_doc_dcont_features, candi_doc_qdcont_features,
           candi_doc_qdiscrete_features, candi_doc_ddiscrete_features,
           q_tables, d_tables, Wqc, bqc, Wdc, bdc, Wqd, bqd,
           Wqdisc, bqdisc, Wddisc, bddisc, Wattn, battn, Wfin, bfin):
  idxs = candi_doc_idxs
  ratings = candi_doc_ratings
  qcont = candi_doc_qcont_features
  dcontF = candi_doc_dcont_features.reshape(POS, -1)
  qdcontF = candi_doc_qdcont_features.reshape(POS, -1)
  qdisc = candi_doc_qdiscrete_features.astype(jnp.int32)
  ddisc = candi_doc_ddiscrete_features.astype(jnp.int32)
  ddiscF = ddisc.reshape(POS, 7)
  ddiscT = jnp.transpose(ddisc, (2, 0, 1)).reshape(7 * POS)
  qdiscT = jnp.transpose(qdisc, (1, 0)).reshape(7 * B)

  gathered = _sc_gather(tuple(d_tables), tuple(q_tables), ddiscT, qdiscT)
  d_embs = gathered[0:7]
  q_embs = gathered[7:14]

  qoff, doff = [0], [0]
  for d in QDIMS:
    qoff.append(qoff[-1] + d)
  for d in DDIMS:
    doff.append(doff[-1] + d)
  wqs = [Wqdisc[qoff[f]:qoff[f + 1]] for f in range(7)]
  wds = [Wddisc[doff[f]:doff[f + 1]] for f in range(7)]
  r2 = lambda v: v.reshape(1, -1)

  grid_spec = pl.GridSpec(
      grid=(GRID,),
      in_specs=[
          pl.BlockSpec((BB, C), lambda i: (i, 0)),
          pl.BlockSpec((BB, C), lambda i: (i, 0)),
          pl.BlockSpec((BB, 15), lambda i: (i, 0)),
          pl.BlockSpec((BB, 7), lambda i: (i, 0)),
          pl.BlockSpec((BB * C, 25), lambda i: (i, 0)),
          pl.BlockSpec((BB * C, 20), lambda i: (i, 0)),
          pl.BlockSpec((BB * C, 7), lambda i: (i, 0)),
      ]
      + [pl.BlockSpec((BB * C, d), lambda i: (i, 0)) for d in DDIMS]
      + [pl.BlockSpec((BB, d), lambda i: (i, 0)) for d in QDIMS]
      + [pl.BlockSpec(w.shape, lambda i: (0, 0)) for w in
         (Wqc, r2(bqc), Wdc, r2(bdc), Wqd, r2(bqd))]
      + [pl.BlockSpec(w.shape, lambda i: (0, 0)) for w in wqs]
      + [pl.BlockSpec(r2(bqdisc).shape, lambda i: (0, 0))]
      + [pl.BlockSpec(w.shape, lambda i: (0, 0)) for w in wds]
      + [pl.BlockSpec(r2(bddisc).shape, lambda i: (0, 0))]
      + [pl.BlockSpec((64, 128), lambda i: (0, 0)),
         pl.BlockSpec((64, 128), lambda i: (0, 0)),
         pl.BlockSpec((128, 128), lambda i: (0, 0)),
         pl.BlockSpec((1, 128), lambda i: (0, 0)),
         pl.BlockSpec((1, 128), lambda i: (0, 0)),
         pl.BlockSpec((1, 1), lambda i: (0, 0))],
      out_specs=pl.BlockSpec((1, 1), lambda i: (0, 0)),
  )

  res = pl.pallas_call(
      _tc_body,
      grid_spec=grid_spec,
      out_shape=jax.ShapeDtypeStruct((1, 1), jnp.float32),
      compiler_params=pltpu.CompilerParams(
          dimension_semantics=("arbitrary",)),
  )(idxs, ratings, qcont, qdisc, dcontF, qdcontF, ddiscF,
    *d_embs, *q_embs,
    Wqc, r2(bqc), Wdc, r2(bdc), Wqd, r2(bqd),
    *wqs, r2(bqdisc), *wds, r2(bddisc),
    Wattn[:64], Wattn[64:], Wattn, r2(battn), Wfin.reshape(1, -1),
    bfin.reshape(1, 1))
  return res[0, 0]


# speedup vs baseline: 2.5966x; 1.1105x over previous
import functools

import jax
import jax.numpy as jnp
from jax import lax
from jax.experimental import pallas as pl
from jax.experimental.pallas import tpu as pltpu
from jax.experimental.pallas import tpu_sc as plsc

B, C = 1024, 50
POS = B * C
QDIMS = (10, 10, 10, 10, 10, 10, 30)
DDIMS = (10, 10, 10, 10, 10, 5, 10)
SCF = (4, 5, 6)
NW = 32
DPW = POS // NW
QPW = B // NW
BB = 32
GRID = B // BB


def _sc_gather(d_tables, q_tables, ddiscT, qdiscT):
  mesh = plsc.VectorSubcoreMesh(core_axis_name="c", subcore_axis_name="s")
  nd = len(SCF)
  ddims = [DDIMS[f] for f in SCF]
  out_type = (
      [jax.ShapeDtypeStruct((POS, d), jnp.float32) for d in ddims]
      + [jax.ShapeDtypeStruct((B, d), jnp.float32) for d in QDIMS]
  )
  scratch_types = (
      [pltpu.VMEM((DPW,), jnp.int32) for _ in range(nd)]
      + [pltpu.VMEM((DPW, d), jnp.float32) for d in ddims]
      + [pltpu.VMEM((QPW,), jnp.int32) for _ in range(7)]
      + [pltpu.VMEM((QPW, d), jnp.float32) for d in QDIMS]
      + [pltpu.SemaphoreType.DMA]
  )

  @functools.partial(pl.kernel, mesh=mesh, out_type=out_type,
                     scratch_types=scratch_types,
                     compiler_params=pltpu.CompilerParams(
                         use_tc_tiling_on_sc=False))
  def gather_kernel(*refs):
    it = iter(range(len(refs)))
    dt = [refs[next(it)] for _ in range(nd)]
    qt = [refs[next(it)] for _ in range(7)]
    ddisc_ref = refs[next(it)]
    qdisc_ref = refs[next(it)]
    douts = [refs[next(it)] for _ in range(nd)]
    qouts = [refs[next(it)] for _ in range(7)]
    didx = [refs[next(it)] for _ in range(nd)]
    dbuf = [refs[next(it)] for _ in range(nd)]
    qidx = [refs[next(it)] for _ in range(7)]
    qbuf = [refs[next(it)] for _ in range(7)]
    sem = refs[next(it)]

    wid = lax.axis_index("s") * 2 + lax.axis_index("c")
    dbase = wid * DPW
    qbase = wid * QPW

    for f in range(nd):
      pltpu.sync_copy(ddisc_ref.at[pl.ds(f * POS + dbase, DPW)], didx[f])
    for f in range(7):
      pltpu.sync_copy(qdisc_ref.at[pl.ds(f * B + qbase, QPW)], qidx[f])

    n_full = DPW // 128
    rem = DPW - n_full * 128
    for f in range(nd):
      def chunk(j, carry, f=f):
        off = pl.multiple_of(j * 128, 128)
        pltpu.async_copy(dt[f].at[didx[f].at[pl.ds(off, 128)]],
                         dbuf[f].at[pl.ds(off, 128), :], sem)
        return carry
      lax.fori_loop(0, n_full, chunk, 0)
      if rem:
        pltpu.async_copy(dt[f].at[didx[f].at[pl.ds(n_full * 128, rem)]],
                         dbuf[f].at[pl.ds(n_full * 128, rem), :], sem)
    for f in range(7):
      pltpu.async_copy(qt[f].at[qidx[f]], qbuf[f], sem)

    for f in range(nd):
      pltpu.make_async_copy(dt[f].at[didx[f]], dbuf[f], sem).wait()
    for f in range(7):
      pltpu.make_async_copy(qt[f].at[qidx[f]], qbuf[f], sem).wait()

    for f in range(nd):
      pltpu.sync_copy(dbuf[f], douts[f].at[pl.ds(dbase, DPW), :])
    for f in range(7):
      pltpu.sync_copy(qbuf[f], qouts[f].at[pl.ds(qbase, QPW), :])

  return gather_kernel(*d_tables, *q_tables, ddiscT, qdiscT)


def _tc_body(*refs):
  (idxs, ratings, qcont, qdisc, dcontF, qdcontF, ddiscF,
   t0, t1, t2, t3, de4, de5, de6,
   qe0, qe1, qe2, qe3, qe4, qe5, qe6,
   Wqc, bqc, Wdc, bdc, Wqd, bqd,
   wq0, wq1, wq2, wq3, wq4, wq5, wq6, bqdisc,
   wd0, wd1, wd2, wd3, wd4, wd5, wd6, bddisc,
   WattnA, WattnB, Wattn, battn, WfinT, bfin, acc) = refs
  small_t = (t0, t1, t2, t3)
  des = (de4, de5, de6)
  qes = (qe0, qe1, qe2, qe3, qe4, qe5, qe6)
  wqs = (wq0, wq1, wq2, wq3, wq4, wq5, wq6)
  wds = (wd0, wd1, wd2, wd3, wd4, wd5, wd6)
  i = pl.program_id(0)
  f32 = jnp.float32
  dot = lambda a, b: jnp.dot(a, b, preferred_element_type=f32)

  qdisc_i = qdisc[...]
  qacc = bqdisc[...]
  for f in range(7):
    m = (qdisc_i[:, f:f + 1] != 0).astype(f32)
    qacc = qacc + m * dot(qes[f][...], wqs[f][...])
  qdisc_h = jnp.tanh(qacc)
  qcont_h = jnp.tanh(dot(qcont[...], Wqc[...]) + bqc[...])
  u_q8 = jnp.tanh(dot(qcont_h, WattnA[...]) + dot(qdisc_h, WattnB[...])
                  + battn[...])

  ddisc_i = ddiscF[...]
  dacc = bddisc[...]
  for k, f in enumerate(SCF):
    m = (ddisc_i[:, f:f + 1] != 0).astype(f32)
    dacc = dacc + m * dot(des[k][...], wds[f][...])
  for f in range(4):
    proj = dot(small_t[f][...], wds[f][...])
    V = small_t[f].shape[0]
    col = ddisc_i[:, f:f + 1]
    oh = (jnp.broadcast_to(col, (BB * C, V))
          == lax.broadcasted_iota(jnp.int32, (BB * C, V), 1)).astype(f32)
    m = (col != 0).astype(f32)
    dacc = dacc + m * dot(oh, proj)
  ddisc_h = jnp.tanh(dacc)
  dcont_h = jnp.tanh(dot(dcontF[...], Wdc[...]) + bdc[...])
  u_d = jnp.tanh(dot(dcont_h, WattnA[...]) + dot(ddisc_h, WattnB[...])
                 + battn[...])
  qdcont_h = jnp.tanh(dot(qdcontF[...], Wqd[...]) + bqd[...])
  u_qd = jnp.tanh(dot(qdcont_h, Wattn[...]) + battn[...])

  r = lax.broadcasted_iota(jnp.int32, (BB * C, BB), 0) // C
  cix = lax.broadcasted_iota(jnp.int32, (BB * C, BB), 1)
  S = (r == cix).astype(f32)
  u_q = dot(S, u_q8)

  rdot = lambda a, b: jnp.sum(a * b, axis=1, keepdims=True)
  g_qq = rdot(u_q, u_q)
  g_qd = rdot(u_q, u_d)
  g_qqd = rdot(u_q, u_qd)
  g_dd = rdot(u_d, u_d)
  g_dqd = rdot(u_d, u_qd)
  g_qdqd = rdot(u_qd, u_qd)

  s0 = jnp.zeros_like(g_qq)
  s1 = jnp.zeros_like(g_qq)
  s2 = jnp.zeros_like(g_qq)
  for a, b, c in ((g_qq, g_qd, g_qqd), (g_qd, g_dd, g_dqd),
                  (g_qqd, g_dqd, g_qdqd)):
    m = jnp.maximum(jnp.maximum(a, b), c)
    ea = jnp.exp(a - m)
    eb = jnp.exp(b - m)
    ec = jnp.exp(c - m)
    inv = 1.0 / (ea + eb + ec)
    s0 = s0 + ea * inv
    s1 = s1 + eb * inv
    s2 = s2 + ec * inv

  wfin = WfinT[...]
  v_q = rdot(u_q, jnp.broadcast_to(wfin, u_q.shape))
  v_d = rdot(u_d, jnp.broadcast_to(wfin, u_d.shape))
  v_qd = rdot(u_qd, jnp.broadcast_to(wfin, u_qd.shape))
  score = s0 * v_q + s1 * v_d + s2 * v_qd + bfin[...]

  rr = lax.broadcasted_iota(jnp.int32, (BB, BB * C), 0)
  cc = lax.broadcasted_iota(jnp.int32, (BB, BB * C), 1)
  ST = (cc // C == rr).astype(f32)
  pcol = lax.broadcasted_iota(jnp.int32, (BB * C, C), 0) % C
  ccol = lax.broadcasted_iota(jnp.int32, (BB * C, C), 1)
  P = jnp.broadcast_to(score, (BB * C, C)) * (pcol == ccol).astype(f32)
  scores2d = dot(ST, P)

  mask2 = (idxs[...] != 0).astype(f32)
  sc = scores2d * mask2
  mx = jnp.max(sc, axis=1, keepdims=True)
  lse = jnp.log(jnp.sum(jnp.exp(sc - mx), axis=1, keepdims=True))
  lsm = sc - mx - lse
  part = jnp.sum(-lsm * ratings[...].astype(f32) * mask2) * (1.0 / B)

  @pl.when(i == 0)
  def _():
    acc[:, :] = jnp.zeros_like(acc)
  acc[:, :] = acc[:, :] + part.reshape(1, 1)


def kernel(candi_doc_idxs, candi_doc_ratings, candi_doc_qcont_features,
           candi_doc_dcont_features, candi_doc_qdcont_features,
           candi_doc_qdiscrete_features, candi_doc_ddiscrete_features,
           q_tables, d_tables, Wqc, bqc, Wdc, bdc, Wqd, bqd,
           Wqdisc, bqdisc, Wddisc, bddisc, Wattn, battn, Wfin, bfin):
  idxs = candi_doc_idxs
  ratings = candi_doc_ratings
  qcont = candi_doc_qcont_features
  dcontF = candi_doc_dcont_features.reshape(POS, -1)
  qdcontF = candi_doc_qdcont_features.reshape(POS, -1)
  qdisc = candi_doc_qdiscrete_features.astype(jnp.int32)
  ddisc = candi_doc_ddiscrete_features.astype(jnp.int32)
  ddiscF = ddisc.reshape(POS, 7)
  ddiscT = jnp.transpose(ddisc[:, :, SCF[0]:], (2, 0, 1)).reshape(
      len(SCF) * POS)
  qdiscT = jnp.transpose(qdisc, (1, 0)).reshape(7 * B)

  gathered = _sc_gather(tuple(d_tables[f] for f in SCF), tuple(q_tables),
                        ddiscT, qdiscT)
  d_embs = gathered[0:len(SCF)]
  q_embs = gathered[len(SCF):]

  qoff, doff = [0], [0]
  for d in QDIMS:
    qoff.append(qoff[-1] + d)
  for d in DDIMS:
    doff.append(doff[-1] + d)
  wqs = [Wqdisc[qoff[f]:qoff[f + 1]] for f in range(7)]
  wds = [Wddisc[doff[f]:doff[f + 1]] for f in range(7)]
  r2 = lambda v: v.reshape(1, -1)
  full = lambda w: pl.BlockSpec(w.shape, lambda i: (0, 0))

  grid_spec = pl.GridSpec(
      grid=(GRID,),
      in_specs=[
          pl.BlockSpec((BB, C), lambda i: (i, 0)),
          pl.BlockSpec((BB, C), lambda i: (i, 0)),
          pl.BlockSpec((BB, 15), lambda i: (i, 0)),
          pl.BlockSpec((BB, 7), lambda i: (i, 0)),
          pl.BlockSpec((BB * C, 25), lambda i: (i, 0)),
          pl.BlockSpec((BB * C, 20), lambda i: (i, 0)),
          pl.BlockSpec((BB * C, 7), lambda i: (i, 0)),
      ]
      + [full(d_tables[f]) for f in range(4)]
      + [pl.BlockSpec((BB * C, DDIMS[f]), lambda i: (i, 0)) for f in SCF]
      + [pl.BlockSpec((BB, d), lambda i: (i, 0)) for d in QDIMS]
      + [full(w) for w in (Wqc, r2(bqc), Wdc, r2(bdc), Wqd, r2(bqd))]
      + [full(w) for w in wqs] + [full(r2(bqdisc))]
      + [full(w) for w in wds] + [full(r2(bddisc))]
      + [full(Wattn[:64]), full(Wattn[64:]), full(Wattn), full(r2(battn)),
         full(Wfin.reshape(1, -1)), full(bfin.reshape(1, 1))],
      out_specs=pl.BlockSpec((1, 1), lambda i: (0, 0)),
  )

  res = pl.pallas_call(
      _tc_body,
      grid_spec=grid_spec,
      out_shape=jax.ShapeDtypeStruct((1, 1), jnp.float32),
      compiler_params=pltpu.CompilerParams(
          dimension_semantics=("arbitrary",)),
  )(idxs, ratings, qcont, qdisc, dcontF, qdcontF, ddiscF,
    *[d_tables[f] for f in range(4)], *d_embs, *q_embs,
    Wqc, r2(bqc), Wdc, r2(bdc), Wqd, r2(bqd),
    *wqs, r2(bqdisc), *wds, r2(bddisc),
    Wattn[:64], Wattn[64:], Wattn, r2(battn), Wfin.reshape(1, -1),
    bfin.reshape(1, 1))
  return res[0, 0]
